# baseline (speedup 1.0000x reference)
import functools

import jax
import jax.numpy as jnp
from jax import lax
from jax.experimental import pallas as pl
from jax.experimental.pallas import tpu as pltpu
from jax.experimental.pallas import tpu_sc as plsc

NB, NT, NP, D = 4, 32, 196, 768
L = 16
DV = D // L
ROWS = 49
NCHUNK = NP // ROWS
SLAB = NP * D
CHUNK = ROWS * D


def _body(x_hbm, fe_hbm, out_hbm, pe_v, buf_v, _sem):
    c = lax.axis_index("c")
    s = lax.axis_index("s")
    w = s * 2 + c

    pltpu.sync_copy(fe_hbm.at[pl.ds(w * D, D)], pe_v)
    pe_vals = [pe_v[pl.ds(j * L, L)] for j in range(DV)]

    for b in range(NB):
        slab_base = (b * NT) * SLAB + w * SLAB
        for ch in range(NCHUNK):
            base = slab_base + ch * CHUNK
            pltpu.sync_copy(x_hbm.at[pl.ds(base, CHUNK)], buf_v)

            def row_body(r, carry):
                r0 = r * D
                for j in range(DV):
                    sl = pl.ds(r0 + j * L, L)
                    buf_v[sl] = buf_v[sl] + pe_vals[j]
                return carry

            lax.fori_loop(0, ROWS, row_body, 0)
            pltpu.sync_copy(buf_v, out_hbm.at[pl.ds(base, CHUNK)])


@jax.jit
def _run(x, frame_embed):
    mesh = plsc.VectorSubcoreMesh(core_axis_name="c", subcore_axis_name="s")
    k = functools.partial(
        pl.kernel,
        mesh=mesh,
        out_type=jax.ShapeDtypeStruct((NB * NT * NP * D,), jnp.float32),
        scratch_types=[
            pltpu.VMEM((D,), jnp.float32),
            pltpu.VMEM((CHUNK,), jnp.float32),
            pltpu.SemaphoreType.DMA,
        ],
    )(_body)
    out = k(x.reshape(-1), frame_embed.reshape(-1))
    return out.reshape(NB, NT, NP, D)


def kernel(x, frame_embed):
    return _run(x, frame_embed)
